# baseline (speedup 1.0000x reference)
import jax
import jax.numpy as jnp
from jax import lax
from jax.experimental import pallas as pl
from jax.experimental.pallas import tpu as pltpu
from jax.experimental.pallas import tpu_sc as plsc

BATCH = 16384
N_FEAT = 39
TOTAL = BATCH * N_FEAT
NUM_WORKERS = 32
CHUNK = TOTAL // NUM_WORKERS
LANES = 16
UNROLL = 8
N_STEPS = CHUNK // (LANES * UNROLL)


def _encode_vec(v, col):
    is_cat = col >= 13
    vi = v.astype(jnp.int32).astype(jnp.float32)
    ok = (vi == v) & (v >= 0.0) & (v <= 15.0)
    return jnp.where(is_cat, jnp.where(ok, v + 1.0, 0.0), v)


def _sc_body(in_hbm, out_hbm, buf):
    wid = lax.axis_index("s") * 2 + lax.axis_index("c")
    base = wid * CHUNK
    pltpu.sync_copy(in_hbm.at[pl.ds(base, CHUNK)], buf)

    iota = lax.iota(jnp.int32, LANES)

    def step(j, carry):
        for k in range(UNROLL):
            off = (j * UNROLL + k) * LANES
            v = buf[pl.ds(off, LANES)]
            col = (base + off + iota) % N_FEAT
            buf[pl.ds(off, LANES)] = _encode_vec(v, col)
        return carry

    lax.fori_loop(0, N_STEPS, step, 0)
    pltpu.sync_copy(buf, out_hbm.at[pl.ds(base, CHUNK)])


@jax.jit
def _sc_encode(flat):
    k = pl.kernel(
        _sc_body,
        out_type=jax.ShapeDtypeStruct((TOTAL,), jnp.float32),
        mesh=plsc.VectorSubcoreMesh(core_axis_name="c", subcore_axis_name="s"),
        scratch_types=[pltpu.VMEM((CHUNK,), jnp.float32)],
    )
    return k(flat)


def kernel(inputs):
    flat = inputs.reshape(TOTAL)
    return _sc_encode(flat).reshape(BATCH, N_FEAT)
